# baseline (speedup 1.0000x reference)
import functools

import jax
import jax.numpy as jnp
from jax import lax
from jax.experimental import pallas as pl
from jax.experimental.pallas import tpu as pltpu
from jax.experimental.pallas import tpu_sc as plsc

_NC = 2
_NS = 16
_NW = _NC * _NS
_K = 128
_CG = 8


def _deg_kernel(dstg, ones_rows, zdeg, *, nch, nrd, rptd):
    mesh = plsc.VectorSubcoreMesh(core_axis_name="c", subcore_axis_name="s")

    @functools.partial(
        pl.kernel,
        mesh=mesh,
        out_type=jax.ShapeDtypeStruct((_NC * nrd, 128), jnp.float32),
        scratch_types=[
            pltpu.VMEM((nch, _K), jnp.int32),
            pltpu.VMEM((_K, 128), jnp.float32),
            pltpu.VMEM_SHARED((nrd, 128), jnp.float32),
            pltpu.SemaphoreType.DMA,
        ],
    )
    def k(dst_h, ones_h, zd_h, out_h, didx, ones_v, deg, dsem):
        c = lax.axis_index("c")
        s = lax.axis_index("s")
        w = c * _NS + s
        pltpu.sync_copy(zd_h, deg.at[pl.ds(s * rptd, rptd)])
        pltpu.sync_copy(ones_h, ones_v)
        pltpu.sync_copy(dst_h.at[pl.ds(w * nch, nch)], didx)
        plsc.subcore_barrier()

        def body(r, carry):
            base = r * 8
            for b in range(8):
                pltpu.async_copy(ones_v, deg.at[didx.at[base + b]], dsem,
                                 add=True)
            for b in range(8):
                pltpu.make_async_copy(ones_v, deg.at[didx.at[base + b]],
                                      dsem).wait()
            return carry

        lax.fori_loop(0, nch // 8, body, 0)
        plsc.subcore_barrier()
        pltpu.sync_copy(deg.at[pl.ds(s * rptd, rptd)],
                        out_h.at[pl.ds(c * nrd + s * rptd, rptd)])

    return k(dstg, ones_rows, zdeg)


def _agg_kernel(y, srcg, dstg, zrows, *, dd, ngw, nr, rpt):
    ng0, ng1 = ngw
    mesh = plsc.VectorSubcoreMesh(core_axis_name="c", subcore_axis_name="s")

    @functools.partial(
        pl.kernel,
        mesh=mesh,
        out_type=jax.ShapeDtypeStruct((_NC * nr, dd), jnp.float32),
        scratch_types=[
            pltpu.VMEM((2, _CG, _K), jnp.int32),
            pltpu.VMEM((2, _CG, _K), jnp.int32),
            pltpu.VMEM((_K, dd), jnp.float32),
            pltpu.VMEM((_K, dd), jnp.float32),
            pltpu.SemaphoreType.DMA,
            pltpu.SemaphoreType.DMA,
            pltpu.SemaphoreType.DMA,
            pltpu.SemaphoreType.DMA,
            pltpu.SemaphoreType.DMA,
            pltpu.SemaphoreType.DMA,
            pltpu.VMEM_SHARED((nr, dd), jnp.float32),
        ],
    )
    def k(y_h, src_h, dst_h, z_h, out_h, sidxb, didxb, r0, r1,
          g0, g1, s0, s1, isem_s, isem_d, acc):
        rows = (r0, r1)
        gsem = (g0, g1)
        ssem = (s0, s1)
        c = lax.axis_index("c")
        s = lax.axis_index("s")
        ng = jnp.where(c == 0, ng0, ng1)
        base = jnp.where(c == 0, s * (ng0 * _CG),
                         _NS * (ng0 * _CG) + s * (ng1 * _CG))
        pltpu.sync_copy(z_h, acc.at[pl.ds(s * rpt, rpt)])
        plsc.subcore_barrier()

        def group_body(g, carry):
            p = lax.rem(g, 2)
            for r in range(_CG // 2):
                for b in range(2):
                    cj = 2 * r + b
                    pltpu.make_async_copy(y_h.at[sidxb.at[p, cj]], rows[b],
                                          gsem[b]).wait()
                    pltpu.async_copy(rows[b], acc.at[didxb.at[p, cj]],
                                     ssem[b], add=True)
                if r < _CG // 2 - 1:
                    for b in range(2):
                        cj = 2 * r + b
                        pltpu.make_async_copy(rows[b],
                                              acc.at[didxb.at[p, cj]],
                                              ssem[b]).wait()
                        pltpu.async_copy(y_h.at[sidxb.at[p, 2 * r + 2 + b]],
                                         rows[b], gsem[b])
                else:
                    pltpu.make_async_copy(src_h.at[pl.ds(base, _CG)],
                                          sidxb.at[1 - p], isem_s).wait()
                    pltpu.make_async_copy(dst_h.at[pl.ds(base, _CG)],
                                          didxb.at[1 - p], isem_d).wait()
                    for b in range(2):
                        cj = 2 * r + b
                        pltpu.make_async_copy(rows[b],
                                              acc.at[didxb.at[p, cj]],
                                              ssem[b]).wait()
                        pltpu.async_copy(y_h.at[sidxb.at[1 - p, b]],
                                         rows[b], gsem[b])
            nxt = base + jnp.minimum(g + 2, ng - 1) * _CG
            pltpu.async_copy(src_h.at[pl.ds(nxt, _CG)], sidxb.at[p],
                             isem_s)
            pltpu.async_copy(dst_h.at[pl.ds(nxt, _CG)], didxb.at[p],
                             isem_d)
            return carry

        @pl.when(ng > 0)
        def _edge_work():
            pltpu.sync_copy(src_h.at[pl.ds(base, _CG)], sidxb.at[0])
            pltpu.sync_copy(dst_h.at[pl.ds(base, _CG)], didxb.at[0])
            pltpu.async_copy(src_h.at[pl.ds(base + _CG, _CG)], sidxb.at[1],
                             isem_s)
            pltpu.async_copy(dst_h.at[pl.ds(base + _CG, _CG)], didxb.at[1],
                             isem_d)
            for b in range(2):
                pltpu.async_copy(y_h.at[sidxb.at[0, b]], rows[b], gsem[b])

            lax.fori_loop(0, ng - 1, group_body, 0)
            pltpu.make_async_copy(src_h.at[pl.ds(base, _CG)], sidxb.at[0],
                                  isem_s).wait()
            pltpu.make_async_copy(dst_h.at[pl.ds(base, _CG)], didxb.at[0],
                                  isem_d).wait()
            pl_ = 1
            for r in range(_CG // 2):
                for b in range(2):
                    cj = 2 * r + b
                    pltpu.make_async_copy(y_h.at[sidxb.at[pl_, cj]], rows[b],
                                          gsem[b]).wait()
                    pltpu.async_copy(rows[b], acc.at[didxb.at[pl_, cj]],
                                     ssem[b], add=True)
                for b in range(2):
                    cj = 2 * r + b
                    pltpu.make_async_copy(rows[b], acc.at[didxb.at[pl_, cj]],
                                          ssem[b]).wait()
                    if r < _CG // 2 - 1:
                        pltpu.async_copy(y_h.at[sidxb.at[pl_, 2 * r + 2 + b]],
                                         rows[b], gsem[b])

        plsc.subcore_barrier()
        pltpu.sync_copy(acc.at[pl.ds(s * rpt, rpt)],
                        out_h.at[pl.ds(c * nr + s * rpt, rpt)])

    return k(y, srcg, dstg, zrows)


def _mm_scale(x, W, dcol, *, bn=1000):
    n, din = x.shape
    dout = W.shape[1]

    def body(x_ref, w_ref, d_ref, o_ref):
        o_ref[...] = jnp.dot(x_ref[...], w_ref[...],
                             preferred_element_type=jnp.float32) * d_ref[...]

    return pl.pallas_call(
        body,
        grid=(n // bn,),
        in_specs=[
            pl.BlockSpec((bn, din), lambda i: (i, 0)),
            pl.BlockSpec((din, dout), lambda i: (0, 0)),
            pl.BlockSpec((bn, 1), lambda i: (i, 0)),
        ],
        out_specs=pl.BlockSpec((bn, dout), lambda i: (i, 0)),
        out_shape=jax.ShapeDtypeStruct((n, dout), jnp.float32),
    )(x, W, dcol)


def _layer_epilogue_mm(accp, y, dcol, b, W, *, bn=1000):
    n, dd = y.shape
    dout = W.shape[1]

    def body(a_ref, y_ref, d_ref, b_ref, w_ref, xl_ref, yn_ref):
        a = a_ref[...]
        dv = d_ref[...]
        xl = jnp.maximum(dv * (a[0] + a[1] + y_ref[...]) + b_ref[...], 0.0)
        xl_ref[...] = xl
        yn_ref[...] = jnp.dot(xl, w_ref[...],
                              preferred_element_type=jnp.float32) * dv

    return pl.pallas_call(
        body,
        grid=(n // bn,),
        in_specs=[
            pl.BlockSpec((2, bn, dd), lambda i: (0, i, 0)),
            pl.BlockSpec((bn, dd), lambda i: (i, 0)),
            pl.BlockSpec((bn, 1), lambda i: (i, 0)),
            pl.BlockSpec((1, dd), lambda i: (0, 0)),
            pl.BlockSpec((dd, dout), lambda i: (0, 0)),
        ],
        out_specs=[
            pl.BlockSpec((bn, dd), lambda i: (i, 0)),
            pl.BlockSpec((bn, dout), lambda i: (i, 0)),
        ],
        out_shape=[
            jax.ShapeDtypeStruct((n, dd), jnp.float32),
            jax.ShapeDtypeStruct((n, dout), jnp.float32),
        ],
    )(accp, y, dcol, b, W)


def _final_mm(accp, y2, dcol, b2, x1, W_out, b_out, *, bn=1000):
    n, dd = y2.shape
    dout = W_out.shape[1]

    def body(a_ref, y_ref, d_ref, b_ref, x1_ref, w_ref, bo_ref, o_ref):
        a = a_ref[...]
        x2 = jnp.maximum(
            d_ref[...] * (a[0] + a[1] + y_ref[...]) + b_ref[...], 0.0)
        o_ref[...] = jnp.dot(x1_ref[...] + x2, w_ref[...],
                             preferred_element_type=jnp.float32) + bo_ref[...]

    return pl.pallas_call(
        body,
        grid=(n // bn,),
        in_specs=[
            pl.BlockSpec((2, bn, dd), lambda i: (0, i, 0)),
            pl.BlockSpec((bn, dd), lambda i: (i, 0)),
            pl.BlockSpec((bn, 1), lambda i: (i, 0)),
            pl.BlockSpec((1, dd), lambda i: (0, 0)),
            pl.BlockSpec((bn, dd), lambda i: (i, 0)),
            pl.BlockSpec((dd, dout), lambda i: (0, 0)),
            pl.BlockSpec((1, dout), lambda i: (0, 0)),
        ],
        out_specs=pl.BlockSpec((bn, dout), lambda i: (i, 0)),
        out_shape=jax.ShapeDtypeStruct((n, dout), jnp.float32),
    )(accp, y2, dcol, b2, x1, W_out, b_out)


def kernel(x, edge_index, W1, b1, W2, b2, W_out, b_out):
    n, _ = x.shape
    d_hid = W1.shape[1]
    e = edge_index.shape[1]
    src = edge_index[0]
    dst = edge_index[1]

    nch = -(-e // (_NW * _K))
    nch = ((nch + 7) // 8) * 8
    totc = _NW * nch
    pad = totc * _K - e
    srcg = jnp.concatenate([src, jnp.zeros((pad,), src.dtype)]).reshape(
        totc, _K)
    dstg = jnp.concatenate([dst, jnp.full((pad,), n, dst.dtype)]).reshape(
        totc, _K)
    tg = totc // (_NS * _CG)
    ng1 = 0
    ng0 = tg - ng1

    rptd = ((-(-(n + 1) // _NS) + 7) // 8) * 8
    nrd = rptd * _NS
    ones_rows = jnp.ones((_K, 128), jnp.float32)
    zdeg = jnp.zeros((rptd, 128), jnp.float32)
    degp = _deg_kernel(dstg, ones_rows, zdeg, nch=nch, nrd=nrd, rptd=rptd)
    degp = degp.reshape(_NC, nrd, 128)
    deg = degp[0, :n, 0] + degp[1, :n, 0] + 1.0
    dcol = lax.rsqrt(deg)[:, None]

    rpt = rptd
    nr = nrd
    zrows = jnp.zeros((rpt, d_hid), jnp.float32)

    y1 = _mm_scale(x, W1, dcol)
    acc1 = _agg_kernel(y1, srcg, dstg, zrows, dd=d_hid, ngw=(ng0, ng1),
                       nr=nr, rpt=rpt).reshape(_NC, nr, d_hid)[:, :n]
    x1, y2 = _layer_epilogue_mm(acc1, y1, dcol, b1.reshape(1, -1), W2)
    acc2 = _agg_kernel(y2, srcg, dstg, zrows, dd=d_hid, ngw=(ng0, ng1),
                       nr=nr, rpt=rpt).reshape(_NC, nr, d_hid)[:, :n]
    return _final_mm(acc2, y2, dcol, b2.reshape(1, -1), x1, W_out,
                     b_out.reshape(1, -1))


# speedup vs baseline: 1.2856x; 1.2856x over previous
import functools

import jax
import jax.numpy as jnp
from jax import lax
from jax.experimental import pallas as pl
from jax.experimental.pallas import tpu as pltpu
from jax.experimental.pallas import tpu_sc as plsc

_NC = 2
_NS = 16
_NW = _NC * _NS
_K = 128
_CG = 8


def _deg_kernel(dstg, ones_rows, zdeg, *, nch, nrd, rptd):
    mesh = plsc.VectorSubcoreMesh(core_axis_name="c", subcore_axis_name="s")

    @functools.partial(
        pl.kernel,
        mesh=mesh,
        out_type=jax.ShapeDtypeStruct((_NC * nrd, 128), jnp.float32),
        scratch_types=[
            pltpu.VMEM((nch, _K), jnp.int32),
            pltpu.VMEM((_K, 128), jnp.float32),
            pltpu.VMEM_SHARED((nrd, 128), jnp.float32),
            pltpu.SemaphoreType.DMA,
        ],
    )
    def k(dst_h, ones_h, zd_h, out_h, didx, ones_v, deg, dsem):
        c = lax.axis_index("c")
        s = lax.axis_index("s")
        w = c * _NS + s
        pltpu.sync_copy(zd_h, deg.at[pl.ds(s * rptd, rptd)])
        pltpu.sync_copy(ones_h, ones_v)
        pltpu.sync_copy(dst_h.at[pl.ds(w * nch, nch)], didx)
        plsc.subcore_barrier()

        def body(r, carry):
            base = r * 8
            for b in range(8):
                pltpu.async_copy(ones_v, deg.at[didx.at[base + b]], dsem,
                                 add=True)
            for b in range(8):
                pltpu.make_async_copy(ones_v, deg.at[didx.at[base + b]],
                                      dsem).wait()
            return carry

        lax.fori_loop(0, nch // 8, body, 0)
        plsc.subcore_barrier()
        pltpu.sync_copy(deg.at[pl.ds(s * rptd, rptd)],
                        out_h.at[pl.ds(c * nrd + s * rptd, rptd)])

    return k(dstg, ones_rows, zdeg)


def _agg_kernel(y, srcg, dstg, zrows, *, dd, ngw, nr, rpt):
    ng0, ng1 = ngw
    mesh = plsc.VectorSubcoreMesh(core_axis_name="c", subcore_axis_name="s")

    @functools.partial(
        pl.kernel,
        mesh=mesh,
        out_type=jax.ShapeDtypeStruct((_NC * nr, dd), jnp.float32),
        scratch_types=[
            pltpu.VMEM((2, _CG, _K), jnp.int32),
            pltpu.VMEM((2, _CG, _K), jnp.int32),
            pltpu.VMEM((_K, dd), jnp.float32),
            pltpu.VMEM((_K, dd), jnp.float32),
            pltpu.SemaphoreType.DMA,
            pltpu.SemaphoreType.DMA,
            pltpu.SemaphoreType.DMA,
            pltpu.SemaphoreType.DMA,
            pltpu.SemaphoreType.DMA,
            pltpu.SemaphoreType.DMA,
            pltpu.VMEM_SHARED((nr, dd), jnp.float32),
        ],
    )
    def k(y_h, src_h, dst_h, z_h, out_h, sidxb, didxb, r0, r1,
          g0, g1, s0, s1, isem_s, isem_d, acc):
        rows = (r0, r1)
        gsem = (g0, g1)
        ssem = (s0, s1)
        c = lax.axis_index("c")
        s = lax.axis_index("s")
        ng = jnp.where(c == 0, ng0, ng1)
        base = jnp.where(c == 0, s * (ng0 * _CG),
                         _NS * (ng0 * _CG) + s * (ng1 * _CG))
        pltpu.sync_copy(z_h, acc.at[pl.ds(s * rpt, rpt)])
        plsc.subcore_barrier()

        def group_body(g, carry):
            p = lax.rem(g, 2)
            for r in range(_CG // 2):
                for b in range(2):
                    cj = 2 * r + b
                    pltpu.make_async_copy(y_h.at[sidxb.at[p, cj]], rows[b],
                                          gsem[b]).wait()
                    pltpu.async_copy(rows[b], acc.at[didxb.at[p, cj]],
                                     ssem[b], add=True)
                if r < _CG // 2 - 1:
                    for b in range(2):
                        cj = 2 * r + b
                        pltpu.make_async_copy(rows[b],
                                              acc.at[didxb.at[p, cj]],
                                              ssem[b]).wait()
                        pltpu.async_copy(y_h.at[sidxb.at[p, 2 * r + 2 + b]],
                                         rows[b], gsem[b])
                else:
                    pltpu.make_async_copy(src_h.at[pl.ds(base, _CG)],
                                          sidxb.at[1 - p], isem_s).wait()
                    pltpu.make_async_copy(dst_h.at[pl.ds(base, _CG)],
                                          didxb.at[1 - p], isem_d).wait()
                    for b in range(2):
                        cj = 2 * r + b
                        pltpu.make_async_copy(rows[b],
                                              acc.at[didxb.at[p, cj]],
                                              ssem[b]).wait()
                        pltpu.async_copy(y_h.at[sidxb.at[1 - p, b]],
                                         rows[b], gsem[b])
            nxt = base + jnp.minimum(g + 2, ng - 1) * _CG
            pltpu.async_copy(src_h.at[pl.ds(nxt, _CG)], sidxb.at[p],
                             isem_s)
            pltpu.async_copy(dst_h.at[pl.ds(nxt, _CG)], didxb.at[p],
                             isem_d)
            return carry

        @pl.when(ng > 0)
        def _edge_work():
            pltpu.sync_copy(src_h.at[pl.ds(base, _CG)], sidxb.at[0])
            pltpu.sync_copy(dst_h.at[pl.ds(base, _CG)], didxb.at[0])
            pltpu.async_copy(src_h.at[pl.ds(base + _CG, _CG)], sidxb.at[1],
                             isem_s)
            pltpu.async_copy(dst_h.at[pl.ds(base + _CG, _CG)], didxb.at[1],
                             isem_d)
            for b in range(2):
                pltpu.async_copy(y_h.at[sidxb.at[0, b]], rows[b], gsem[b])

            lax.fori_loop(0, ng - 1, group_body, 0)
            pltpu.make_async_copy(src_h.at[pl.ds(base, _CG)], sidxb.at[0],
                                  isem_s).wait()
            pltpu.make_async_copy(dst_h.at[pl.ds(base, _CG)], didxb.at[0],
                                  isem_d).wait()
            pl_ = 1
            for r in range(_CG // 2):
                for b in range(2):
                    cj = 2 * r + b
                    pltpu.make_async_copy(y_h.at[sidxb.at[pl_, cj]], rows[b],
                                          gsem[b]).wait()
                    pltpu.async_copy(rows[b], acc.at[didxb.at[pl_, cj]],
                                     ssem[b], add=True)
                for b in range(2):
                    cj = 2 * r + b
                    pltpu.make_async_copy(rows[b], acc.at[didxb.at[pl_, cj]],
                                          ssem[b]).wait()
                    if r < _CG // 2 - 1:
                        pltpu.async_copy(y_h.at[sidxb.at[pl_, 2 * r + 2 + b]],
                                         rows[b], gsem[b])

        plsc.subcore_barrier()
        pltpu.sync_copy(acc.at[pl.ds(s * rpt, rpt)],
                        out_h.at[pl.ds(c * nr + s * rpt, rpt)])

    return k(y, srcg, dstg, zrows)


def _mm_scale(x, W, dcol, *, bn=1000):
    n, din = x.shape
    dout = W.shape[1]

    def body(x_ref, w_ref, d_ref, o_ref):
        o_ref[...] = jnp.dot(x_ref[...], w_ref[...],
                             preferred_element_type=jnp.float32) * d_ref[...]

    return pl.pallas_call(
        body,
        grid=(n // bn,),
        in_specs=[
            pl.BlockSpec((bn, din), lambda i: (i, 0)),
            pl.BlockSpec((din, dout), lambda i: (0, 0)),
            pl.BlockSpec((bn, 1), lambda i: (i, 0)),
        ],
        out_specs=pl.BlockSpec((bn, dout), lambda i: (i, 0)),
        out_shape=jax.ShapeDtypeStruct((n, dout), jnp.float32),
    )(x, W, dcol)


def _layer_epilogue_mm(accp, y, dcol, b, W, *, bn=1000):
    n, dd = y.shape
    dout = W.shape[1]

    def body(a_ref, y_ref, d_ref, b_ref, w_ref, xl_ref, yn_ref):
        a = a_ref[...]
        dv = d_ref[...]
        xl = jnp.maximum(dv * (a[0] + a[1] + y_ref[...]) + b_ref[...], 0.0)
        xl_ref[...] = xl
        yn_ref[...] = jnp.dot(xl, w_ref[...],
                              preferred_element_type=jnp.float32) * dv

    return pl.pallas_call(
        body,
        grid=(n // bn,),
        in_specs=[
            pl.BlockSpec((2, bn, dd), lambda i: (0, i, 0)),
            pl.BlockSpec((bn, dd), lambda i: (i, 0)),
            pl.BlockSpec((bn, 1), lambda i: (i, 0)),
            pl.BlockSpec((1, dd), lambda i: (0, 0)),
            pl.BlockSpec((dd, dout), lambda i: (0, 0)),
        ],
        out_specs=[
            pl.BlockSpec((bn, dd), lambda i: (i, 0)),
            pl.BlockSpec((bn, dout), lambda i: (i, 0)),
        ],
        out_shape=[
            jax.ShapeDtypeStruct((n, dd), jnp.float32),
            jax.ShapeDtypeStruct((n, dout), jnp.float32),
        ],
    )(accp, y, dcol, b, W)


def _final_mm(accp, y2, dcol, b2, x1, W_out, b_out, *, bn=1000):
    n, dd = y2.shape
    dout = W_out.shape[1]

    def body(a_ref, y_ref, d_ref, b_ref, x1_ref, w_ref, bo_ref, o_ref):
        a = a_ref[...]
        x2 = jnp.maximum(
            d_ref[...] * (a[0] + a[1] + y_ref[...]) + b_ref[...], 0.0)
        o_ref[...] = jnp.dot(x1_ref[...] + x2, w_ref[...],
                             preferred_element_type=jnp.float32) + bo_ref[...]

    return pl.pallas_call(
        body,
        grid=(n // bn,),
        in_specs=[
            pl.BlockSpec((2, bn, dd), lambda i: (0, i, 0)),
            pl.BlockSpec((bn, dd), lambda i: (i, 0)),
            pl.BlockSpec((bn, 1), lambda i: (i, 0)),
            pl.BlockSpec((1, dd), lambda i: (0, 0)),
            pl.BlockSpec((bn, dd), lambda i: (i, 0)),
            pl.BlockSpec((dd, dout), lambda i: (0, 0)),
            pl.BlockSpec((1, dout), lambda i: (0, 0)),
        ],
        out_specs=pl.BlockSpec((bn, dout), lambda i: (i, 0)),
        out_shape=jax.ShapeDtypeStruct((n, dout), jnp.float32),
    )(accp, y2, dcol, b2, x1, W_out, b_out)


def kernel(x, edge_index, W1, b1, W2, b2, W_out, b_out):
    n, _ = x.shape
    d_hid = W1.shape[1]
    e = edge_index.shape[1]
    src = edge_index[0]
    dst = edge_index[1]

    nch = -(-e // (_NW * _K))
    nch = ((nch + 7) // 8) * 8
    totc = _NW * nch
    pad = totc * _K - e
    srcg = jnp.concatenate([src, jnp.zeros((pad,), src.dtype)]).reshape(
        totc, _K)
    dstg = jnp.concatenate([dst, jnp.full((pad,), n, dst.dtype)]).reshape(
        totc, _K)
    tg = totc // (_NS * _CG)
    ng1 = max(2, ((3 * tg) // 10) & ~1)
    ng0 = tg - ng1

    rptd = ((-(-(n + 1) // _NS) + 7) // 8) * 8
    nrd = rptd * _NS
    ones_rows = jnp.ones((_K, 128), jnp.float32)
    zdeg = jnp.zeros((rptd, 128), jnp.float32)
    degp = _deg_kernel(dstg, ones_rows, zdeg, nch=nch, nrd=nrd, rptd=rptd)
    degp = degp.reshape(_NC, nrd, 128)
    deg = degp[0, :n, 0] + degp[1, :n, 0] + 1.0
    dcol = lax.rsqrt(deg)[:, None]

    rpt = rptd
    nr = nrd
    zrows = jnp.zeros((rpt, d_hid), jnp.float32)

    y1 = _mm_scale(x, W1, dcol)
    acc1 = _agg_kernel(y1, srcg, dstg, zrows, dd=d_hid, ngw=(ng0, ng1),
                       nr=nr, rpt=rpt).reshape(_NC, nr, d_hid)[:, :n]
    x1, y2 = _layer_epilogue_mm(acc1, y1, dcol, b1.reshape(1, -1), W2)
    acc2 = _agg_kernel(y2, srcg, dstg, zrows, dd=d_hid, ngw=(ng0, ng1),
                       nr=nr, rpt=rpt).reshape(_NC, nr, d_hid)[:, :n]
    return _final_mm(acc2, y2, dcol, b2.reshape(1, -1), x1, W_out,
                     b_out.reshape(1, -1))


# speedup vs baseline: 1.4935x; 1.1617x over previous
import functools

import jax
import jax.numpy as jnp
from jax import lax
from jax.experimental import pallas as pl
from jax.experimental.pallas import tpu as pltpu
from jax.experimental.pallas import tpu_sc as plsc

_NC = 2
_NS = 16
_NW = _NC * _NS
_K = 128
_CG = 8


def _deg_kernel(dstg, ones_rows, zdeg, *, nch, nrd, rptd):
    mesh = plsc.VectorSubcoreMesh(core_axis_name="c", subcore_axis_name="s")

    @functools.partial(
        pl.kernel,
        mesh=mesh,
        out_type=jax.ShapeDtypeStruct((_NC * nrd, 128), jnp.float32),
        scratch_types=[
            pltpu.VMEM((nch, _K), jnp.int32),
            pltpu.VMEM((_K, 128), jnp.float32),
            pltpu.VMEM_SHARED((nrd, 128), jnp.float32),
            pltpu.SemaphoreType.DMA,
        ],
    )
    def k(dst_h, ones_h, zd_h, out_h, didx, ones_v, deg, dsem):
        c = lax.axis_index("c")
        s = lax.axis_index("s")
        w = c * _NS + s
        pltpu.sync_copy(zd_h, deg.at[pl.ds(s * rptd, rptd)])
        pltpu.sync_copy(ones_h, ones_v)
        pltpu.sync_copy(dst_h.at[pl.ds(w * nch, nch)], didx)
        plsc.subcore_barrier()

        def body(r, carry):
            base = r * 8
            for b in range(8):
                pltpu.async_copy(ones_v, deg.at[didx.at[base + b]], dsem,
                                 add=True)
            for b in range(8):
                pltpu.make_async_copy(ones_v, deg.at[didx.at[base + b]],
                                      dsem).wait()
            return carry

        lax.fori_loop(0, nch // 8, body, 0)
        plsc.subcore_barrier()
        pltpu.sync_copy(deg.at[pl.ds(s * rptd, rptd)],
                        out_h.at[pl.ds(c * nrd + s * rptd, rptd)])

    return k(dstg, ones_rows, zdeg)


def _agg_kernel(y, srcg, dstg, zrows, *, dd, ngw, nr, rpt):
    ng0, ng1 = ngw
    mesh = plsc.VectorSubcoreMesh(core_axis_name="c", subcore_axis_name="s")

    @functools.partial(
        pl.kernel,
        mesh=mesh,
        out_type=jax.ShapeDtypeStruct((_NC * nr, dd), jnp.float32),
        scratch_types=[
            pltpu.VMEM((2, _CG, _K), jnp.int32),
            pltpu.VMEM((2, _CG, _K), jnp.int32),
            pltpu.VMEM((_K, dd), jnp.float32),
            pltpu.VMEM((_K, dd), jnp.float32),
            pltpu.SemaphoreType.DMA,
            pltpu.SemaphoreType.DMA,
            pltpu.SemaphoreType.DMA,
            pltpu.SemaphoreType.DMA,
            pltpu.SemaphoreType.DMA,
            pltpu.SemaphoreType.DMA,
            pltpu.VMEM_SHARED((nr, dd), jnp.float32),
        ],
    )
    def k(y_h, src_h, dst_h, z_h, out_h, sidxb, didxb, r0, r1,
          g0, g1, s0, s1, isem_s, isem_d, acc):
        rows = (r0, r1)
        gsem = (g0, g1)
        ssem = (s0, s1)
        c = lax.axis_index("c")
        s = lax.axis_index("s")
        ng = jnp.where(c == 0, ng0, ng1)
        base = jnp.where(c == 0, s * (ng0 * _CG),
                         _NS * (ng0 * _CG) + s * (ng1 * _CG))
        pltpu.sync_copy(z_h, acc.at[pl.ds(s * rpt, rpt)])
        plsc.subcore_barrier()

        def group_body(g, carry):
            p = lax.rem(g, 2)
            for r in range(_CG // 2):
                for b in range(2):
                    cj = 2 * r + b
                    pltpu.make_async_copy(y_h.at[sidxb.at[p, cj]], rows[b],
                                          gsem[b]).wait()
                    pltpu.async_copy(rows[b], acc.at[didxb.at[p, cj]],
                                     ssem[b], add=True)
                if r < _CG // 2 - 1:
                    for b in range(2):
                        cj = 2 * r + b
                        pltpu.make_async_copy(rows[b],
                                              acc.at[didxb.at[p, cj]],
                                              ssem[b]).wait()
                        pltpu.async_copy(y_h.at[sidxb.at[p, 2 * r + 2 + b]],
                                         rows[b], gsem[b])
                else:
                    pltpu.make_async_copy(src_h.at[pl.ds(base, _CG)],
                                          sidxb.at[1 - p], isem_s).wait()
                    pltpu.make_async_copy(dst_h.at[pl.ds(base, _CG)],
                                          didxb.at[1 - p], isem_d).wait()
                    for b in range(2):
                        cj = 2 * r + b
                        pltpu.make_async_copy(rows[b],
                                              acc.at[didxb.at[p, cj]],
                                              ssem[b]).wait()
                        pltpu.async_copy(y_h.at[sidxb.at[1 - p, b]],
                                         rows[b], gsem[b])
            nxt = base + jnp.minimum(g + 2, ng - 1) * _CG
            pltpu.async_copy(src_h.at[pl.ds(nxt, _CG)], sidxb.at[p],
                             isem_s)
            pltpu.async_copy(dst_h.at[pl.ds(nxt, _CG)], didxb.at[p],
                             isem_d)
            return carry

        @pl.when(ng > 0)
        def _edge_work():
            pltpu.sync_copy(src_h.at[pl.ds(base, _CG)], sidxb.at[0])
            pltpu.sync_copy(dst_h.at[pl.ds(base, _CG)], didxb.at[0])
            pltpu.async_copy(src_h.at[pl.ds(base + _CG, _CG)], sidxb.at[1],
                             isem_s)
            pltpu.async_copy(dst_h.at[pl.ds(base + _CG, _CG)], didxb.at[1],
                             isem_d)
            for b in range(2):
                pltpu.async_copy(y_h.at[sidxb.at[0, b]], rows[b], gsem[b])

            lax.fori_loop(0, ng - 1, group_body, 0)
            pltpu.make_async_copy(src_h.at[pl.ds(base, _CG)], sidxb.at[0],
                                  isem_s).wait()
            pltpu.make_async_copy(dst_h.at[pl.ds(base, _CG)], didxb.at[0],
                                  isem_d).wait()
            pl_ = 1
            for r in range(_CG // 2):
                for b in range(2):
                    cj = 2 * r + b
                    pltpu.make_async_copy(y_h.at[sidxb.at[pl_, cj]], rows[b],
                                          gsem[b]).wait()
                    pltpu.async_copy(rows[b], acc.at[didxb.at[pl_, cj]],
                                     ssem[b], add=True)
                for b in range(2):
                    cj = 2 * r + b
                    pltpu.make_async_copy(rows[b], acc.at[didxb.at[pl_, cj]],
                                          ssem[b]).wait()
                    if r < _CG // 2 - 1:
                        pltpu.async_copy(y_h.at[sidxb.at[pl_, 2 * r + 2 + b]],
                                         rows[b], gsem[b])

        plsc.subcore_barrier()
        pltpu.sync_copy(acc.at[pl.ds(s * rpt, rpt)],
                        out_h.at[pl.ds(c * nr + s * rpt, rpt)])

    return k(y, srcg, dstg, zrows)


def _mm_scale(x, W, dcol, *, bn=1000):
    n, din = x.shape
    dout = W.shape[1]

    def body(x_ref, w_ref, d_ref, o_ref):
        o_ref[...] = jnp.dot(x_ref[...], w_ref[...],
                             preferred_element_type=jnp.float32) * d_ref[...]

    return pl.pallas_call(
        body,
        grid=(n // bn,),
        in_specs=[
            pl.BlockSpec((bn, din), lambda i: (i, 0)),
            pl.BlockSpec((din, dout), lambda i: (0, 0)),
            pl.BlockSpec((bn, 1), lambda i: (i, 0)),
        ],
        out_specs=pl.BlockSpec((bn, dout), lambda i: (i, 0)),
        out_shape=jax.ShapeDtypeStruct((n, dout), jnp.float32),
    )(x, W, dcol)


def _layer_epilogue_mm(accp, y, dcol, b, W, *, bn=1000):
    n, dd = y.shape
    dout = W.shape[1]

    def body(a_ref, y_ref, d_ref, b_ref, w_ref, xl_ref, yn_ref):
        a = a_ref[...]
        dv = d_ref[...]
        xl = jnp.maximum(dv * (a[0] + a[1] + y_ref[...]) + b_ref[...], 0.0)
        xl_ref[...] = xl
        yn_ref[...] = jnp.dot(xl, w_ref[...],
                              preferred_element_type=jnp.float32) * dv

    return pl.pallas_call(
        body,
        grid=(n // bn,),
        in_specs=[
            pl.BlockSpec((2, bn, dd), lambda i: (0, i, 0)),
            pl.BlockSpec((bn, dd), lambda i: (i, 0)),
            pl.BlockSpec((bn, 1), lambda i: (i, 0)),
            pl.BlockSpec((1, dd), lambda i: (0, 0)),
            pl.BlockSpec((dd, dout), lambda i: (0, 0)),
        ],
        out_specs=[
            pl.BlockSpec((bn, dd), lambda i: (i, 0)),
            pl.BlockSpec((bn, dout), lambda i: (i, 0)),
        ],
        out_shape=[
            jax.ShapeDtypeStruct((n, dd), jnp.float32),
            jax.ShapeDtypeStruct((n, dout), jnp.float32),
        ],
    )(accp, y, dcol, b, W)


def _final_mm(accp, y2, dcol, b2, x1, W_out, b_out, *, bn=1000):
    n, dd = y2.shape
    dout = W_out.shape[1]

    def body(a_ref, y_ref, d_ref, b_ref, x1_ref, w_ref, bo_ref, o_ref):
        a = a_ref[...]
        x2 = jnp.maximum(
            d_ref[...] * (a[0] + a[1] + y_ref[...]) + b_ref[...], 0.0)
        o_ref[...] = jnp.dot(x1_ref[...] + x2, w_ref[...],
                             preferred_element_type=jnp.float32) + bo_ref[...]

    return pl.pallas_call(
        body,
        grid=(n // bn,),
        in_specs=[
            pl.BlockSpec((2, bn, dd), lambda i: (0, i, 0)),
            pl.BlockSpec((bn, dd), lambda i: (i, 0)),
            pl.BlockSpec((bn, 1), lambda i: (i, 0)),
            pl.BlockSpec((1, dd), lambda i: (0, 0)),
            pl.BlockSpec((bn, dd), lambda i: (i, 0)),
            pl.BlockSpec((dd, dout), lambda i: (0, 0)),
            pl.BlockSpec((1, dout), lambda i: (0, 0)),
        ],
        out_specs=pl.BlockSpec((bn, dout), lambda i: (i, 0)),
        out_shape=jax.ShapeDtypeStruct((n, dout), jnp.float32),
    )(accp, y2, dcol, b2, x1, W_out, b_out)


def kernel(x, edge_index, W1, b1, W2, b2, W_out, b_out):
    n, _ = x.shape
    d_hid = W1.shape[1]
    e = edge_index.shape[1]
    src = edge_index[0]
    dst = edge_index[1]

    nch = -(-e // (_NW * _K))
    nch = ((nch + 7) // 8) * 8
    totc = _NW * nch
    pad = totc * _K - e
    srcg = jnp.concatenate([src, jnp.zeros((pad,), src.dtype)]).reshape(
        totc, _K)
    dstg = jnp.concatenate([dst, jnp.full((pad,), n, dst.dtype)]).reshape(
        totc, _K)
    tg = totc // (_NS * _CG)
    ng1 = max(2, (tg // 10) & ~1)
    ng0 = tg - ng1

    rptd = ((-(-(n + 1) // _NS) + 7) // 8) * 8
    nrd = rptd * _NS
    ones_rows = jnp.ones((_K, 128), jnp.float32)
    zdeg = jnp.zeros((rptd, 128), jnp.float32)
    degp = _deg_kernel(dstg, ones_rows, zdeg, nch=nch, nrd=nrd, rptd=rptd)
    degp = degp.reshape(_NC, nrd, 128)
    deg = degp[0, :n, 0] + degp[1, :n, 0] + 1.0
    dcol = lax.rsqrt(deg)[:, None]

    rpt = rptd
    nr = nrd
    zrows = jnp.zeros((rpt, d_hid), jnp.float32)

    y1 = _mm_scale(x, W1, dcol)
    acc1 = _agg_kernel(y1, srcg, dstg, zrows, dd=d_hid, ngw=(ng0, ng1),
                       nr=nr, rpt=rpt).reshape(_NC, nr, d_hid)[:, :n]
    x1, y2 = _layer_epilogue_mm(acc1, y1, dcol, b1.reshape(1, -1), W2)
    acc2 = _agg_kernel(y2, srcg, dstg, zrows, dd=d_hid, ngw=(ng0, ng1),
                       nr=nr, rpt=rpt).reshape(_NC, nr, d_hid)[:, :n]
    return _final_mm(acc2, y2, dcol, b2.reshape(1, -1), x1, W_out,
                     b_out.reshape(1, -1))
